# baseline (speedup 1.0000x reference)
import functools

import jax
import jax.numpy as jnp
from jax import lax
from jax.experimental import pallas as pl
from jax.experimental.pallas import tpu as pltpu
from jax.experimental.pallas import tpu_sc as plsc

BATCH = 16384
EMBED_DIM = 128
NC = 2
NS = 16
NW = NC * NS
B_PER_W = BATCH // NW
CHUNK = 128
NCHUNKS = B_PER_W // CHUNK
SEG = EMBED_DIM // 16


def _mf_kernel(uid_hbm, aid_hbm, ue_hbm, ae_hbm, ub_hbm, ab_hbm, gb_hbm,
               out_hbm,
               uidx_v, aidx_v, ubias_v, abias_v, ue_buf, ae_buf,
               out_buf, gb_v,
               sem0, sem1):
    wid = lax.axis_index("s") * NC + lax.axis_index("c")
    base = wid * B_PER_W
    sems = (sem0, sem1)

    pltpu.sync_copy(uid_hbm.at[pl.ds(wid * NCHUNKS, NCHUNKS)], uidx_v)
    pltpu.sync_copy(aid_hbm.at[pl.ds(wid * NCHUNKS, NCHUNKS)], aidx_v)
    pltpu.sync_copy(ub_hbm.at[pl.ds(wid * NCHUNKS, NCHUNKS)], ubias_v)
    pltpu.sync_copy(ab_hbm.at[pl.ds(wid * NCHUNKS, NCHUNKS)], abias_v)
    pltpu.sync_copy(gb_hbm, gb_v)

    def start(c):
        slot = c % 2
        sem = sems[slot]
        return [
            pltpu.async_copy(ue_hbm.at[uidx_v.at[c]], ue_buf.at[slot], sem),
            pltpu.async_copy(ae_hbm.at[aidx_v.at[c]], ae_buf.at[slot], sem),
        ]

    def compute_chunk(c):
        slot = c % 2
        iota = lax.iota(jnp.int32, 16)
        gb = gb_v[...]

        def group_body(g, carry):
            row0 = g * 16
            tot = gb
            for r in range(16):
                row = row0 + r
                acc = (ue_buf[slot, row, pl.ds(0, 16)] *
                       ae_buf[slot, row, pl.ds(0, 16)])
                for s in range(1, SEG):
                    acc = acc + (ue_buf[slot, row, pl.ds(s * 16, 16)] *
                                 ae_buf[slot, row, pl.ds(s * 16, 16)])
                dot = jnp.sum(acc)
                tot = jnp.where(iota == r, dot, tot)
            tot = tot + ubias_v[c, pl.ds(row0, 16)]
            tot = tot + abias_v[c, pl.ds(row0, 16)]
            out_buf[pl.ds(c * CHUNK + row0, 16)] = tot
            return carry

        lax.fori_loop(0, CHUNK // 16, group_body, 0)

    copies = start(0)
    for c in range(NCHUNKS):
        nxt = start(c + 1) if c + 1 < NCHUNKS else None
        for cp in copies:
            cp.wait()
        compute_chunk(c)
        copies = nxt

    pltpu.sync_copy(out_buf, out_hbm.at[pl.ds(base, B_PER_W)])


def kernel(user_id, anime_id, user_embedding, anime_embedding, user_bias,
           anime_bias, global_bias):
    mesh = plsc.VectorSubcoreMesh(core_axis_name="c", subcore_axis_name="s")
    run = functools.partial(
        pl.kernel,
        mesh=mesh,
        compiler_params=pltpu.CompilerParams(
            needs_layout_passes=False, use_tc_tiling_on_sc=False),
        out_type=jax.ShapeDtypeStruct((BATCH,), jnp.float32),
        scratch_types=[
            pltpu.VMEM((NCHUNKS, CHUNK), jnp.int32),
            pltpu.VMEM((NCHUNKS, CHUNK), jnp.int32),
            pltpu.VMEM((NCHUNKS, CHUNK), jnp.float32),
            pltpu.VMEM((NCHUNKS, CHUNK), jnp.float32),
            pltpu.VMEM((2, CHUNK, EMBED_DIM), jnp.float32),
            pltpu.VMEM((2, CHUNK, EMBED_DIM), jnp.float32),
            pltpu.VMEM((B_PER_W,), jnp.float32),
            pltpu.VMEM((16,), jnp.float32),
            pltpu.SemaphoreType.DMA,
            pltpu.SemaphoreType.DMA,
        ],
    )(_mf_kernel)
    uid2d = user_id.astype(jnp.int32).reshape(BATCH // CHUNK, CHUNK)
    aid2d = anime_id.astype(jnp.int32).reshape(BATCH // CHUNK, CHUNK)
    ub_g = jax.lax.optimization_barrier(
        jnp.take(user_bias, user_id, axis=0)).reshape(BATCH // CHUNK, CHUNK)
    ab_g = jax.lax.optimization_barrier(
        jnp.take(anime_bias, anime_id, axis=0)).reshape(BATCH // CHUNK, CHUNK)
    return run(uid2d, aid2d, user_embedding, anime_embedding, ub_g, ab_g,
               jnp.broadcast_to(global_bias, (16,)))


# speedup vs baseline: 1.6077x; 1.6077x over previous
import functools

import jax
import jax.numpy as jnp
from jax import lax
from jax.experimental import pallas as pl
from jax.experimental.pallas import tpu as pltpu
from jax.experimental.pallas import tpu_sc as plsc

BATCH = 16384
EMBED_DIM = 128
NC = 2
NS = 16
NW = NC * NS
B_PER_W = BATCH // NW
CHUNK = 128
NCHUNKS = B_PER_W // CHUNK
SEG = EMBED_DIM // 16

_COMPILER_PARAMS = pltpu.CompilerParams(
    needs_layout_passes=False, use_tc_tiling_on_sc=False)
_MESH = plsc.VectorSubcoreMesh(core_axis_name="c", subcore_axis_name="s")


def _dots_kernel(uid_hbm, aid_hbm, ue_hbm, ae_hbm,
                 out_hbm,
                 uidx_v, aidx_v, ue_buf, ae_buf, out_buf,
                 sem0, sem1):
    wid = lax.axis_index("s") * NC + lax.axis_index("c")
    sems = (sem0, sem1)

    pltpu.sync_copy(uid_hbm.at[pl.ds(wid * NCHUNKS, NCHUNKS)], uidx_v)
    pltpu.sync_copy(aid_hbm.at[pl.ds(wid * NCHUNKS, NCHUNKS)], aidx_v)

    def start(c):
        slot = c % 2
        sem = sems[slot]
        return [
            pltpu.async_copy(ue_hbm.at[uidx_v.at[c]], ue_buf.at[slot], sem),
            pltpu.async_copy(ae_hbm.at[aidx_v.at[c]], ae_buf.at[slot], sem),
        ]

    def compute_chunk(c):
        slot = c % 2
        iota = lax.iota(jnp.int32, 16)

        def group_body(g, carry):
            row0 = g * 16
            tot = jnp.zeros((16,), jnp.float32)
            for r in range(16):
                row = row0 + r
                acc = (ue_buf[slot, row, pl.ds(0, 16)] *
                       ae_buf[slot, row, pl.ds(0, 16)])
                for s in range(1, SEG):
                    acc = acc + (ue_buf[slot, row, pl.ds(s * 16, 16)] *
                                 ae_buf[slot, row, pl.ds(s * 16, 16)])
                dot = jnp.sum(acc)
                tot = jnp.where(iota == r, dot, tot)
            out_buf[c, pl.ds(row0, 16)] = tot
            return carry

        lax.fori_loop(0, CHUNK // 16, group_body, 0)

    copies = start(0)
    for c in range(NCHUNKS):
        nxt = start(c + 1) if c + 1 < NCHUNKS else None
        for cp in copies:
            cp.wait()
        compute_chunk(c)
        copies = nxt

    pltpu.sync_copy(out_buf, out_hbm.at[pl.ds(wid * NCHUNKS, NCHUNKS)])


def _bias_kernel(uid_hbm, aid_hbm, dots_hbm, ub_hbm, ab_hbm, gb_hbm,
                 out_hbm,
                 uidx_v, aidx_v, dots_v, ub_v, ab_v, out_buf, gb_v,
                 sem):
    wid = lax.axis_index("s") * NC + lax.axis_index("c")
    base = wid * B_PER_W

    pltpu.sync_copy(uid_hbm.at[pl.ds(wid * NCHUNKS, NCHUNKS)], uidx_v)
    pltpu.sync_copy(aid_hbm.at[pl.ds(wid * NCHUNKS, NCHUNKS)], aidx_v)
    pltpu.sync_copy(dots_hbm.at[pl.ds(wid * NCHUNKS, NCHUNKS)], dots_v)
    pltpu.sync_copy(gb_hbm, gb_v)

    copies = []
    for c in range(NCHUNKS):
        copies.append(
            pltpu.async_copy(ub_hbm.at[uidx_v.at[c]], ub_v.at[c], sem))
        copies.append(
            pltpu.async_copy(ab_hbm.at[aidx_v.at[c]], ab_v.at[c], sem))
    for cp in copies:
        cp.wait()

    gb = gb_v[...]

    def body(j, carry):
        c = j // (CHUNK // 16)
        row0 = (j % (CHUNK // 16)) * 16
        sl = pl.ds(row0, 16)
        out_buf[pl.ds(j * 16, 16)] = (dots_v[c, sl] + ub_v[c, sl] +
                                      ab_v[c, sl] + gb)
        return carry

    lax.fori_loop(0, B_PER_W // 16, body, 0)

    pltpu.sync_copy(out_buf, out_hbm.at[pl.ds(base, B_PER_W)])


def kernel(user_id, anime_id, user_embedding, anime_embedding, user_bias,
           anime_bias, global_bias):
    run_dots = functools.partial(
        pl.kernel,
        mesh=_MESH,
        compiler_params=_COMPILER_PARAMS,
        out_type=jax.ShapeDtypeStruct((BATCH // CHUNK, CHUNK), jnp.float32),
        scratch_types=[
            pltpu.VMEM((NCHUNKS, CHUNK), jnp.int32),
            pltpu.VMEM((NCHUNKS, CHUNK), jnp.int32),
            pltpu.VMEM((2, CHUNK, EMBED_DIM), jnp.float32),
            pltpu.VMEM((2, CHUNK, EMBED_DIM), jnp.float32),
            pltpu.VMEM((NCHUNKS, CHUNK), jnp.float32),
            pltpu.SemaphoreType.DMA,
            pltpu.SemaphoreType.DMA,
        ],
    )(_dots_kernel)
    run_bias = functools.partial(
        pl.kernel,
        mesh=_MESH,
        compiler_params=_COMPILER_PARAMS,
        out_type=jax.ShapeDtypeStruct((BATCH,), jnp.float32),
        scratch_types=[
            pltpu.VMEM((NCHUNKS, CHUNK), jnp.int32),
            pltpu.VMEM((NCHUNKS, CHUNK), jnp.int32),
            pltpu.VMEM((NCHUNKS, CHUNK), jnp.float32),
            pltpu.VMEM((NCHUNKS, CHUNK), jnp.float32),
            pltpu.VMEM((NCHUNKS, CHUNK), jnp.float32),
            pltpu.VMEM((B_PER_W,), jnp.float32),
            pltpu.VMEM((16,), jnp.float32),
            pltpu.SemaphoreType.DMA,
        ],
    )(_bias_kernel)

    uid2d = user_id.astype(jnp.int32).reshape(BATCH // CHUNK, CHUNK)
    aid2d = anime_id.astype(jnp.int32).reshape(BATCH // CHUNK, CHUNK)
    ub_flat = user_bias.reshape(-1)
    ab_flat = anime_bias.reshape(-1)
    gb16 = jnp.broadcast_to(global_bias, (16,))

    dots = run_dots(uid2d, aid2d, user_embedding, anime_embedding)
    return run_bias(uid2d, aid2d, dots, ub_flat, ab_flat, gb16)
